# Initial kernel scaffold; baseline (speedup 1.0000x reference)
import jax
import jax.numpy as jnp
from jax.experimental import pallas as pl


def kernel(value, scale, zero_point, flat_indices, rand_int):
    raise NotImplementedError("write your pallas kernel here")


# speedup vs baseline: 4.5988x; 4.5988x over previous
import functools

import jax
import jax.numpy as jnp
from jax import lax
from jax.experimental import pallas as pl
from jax.experimental.pallas import tpu as pltpu
from jax.experimental.pallas import tpu_sc as plsc

NC = 2
NS = 16
NW = NC * NS
LANES = 16
BATCH = 128


def _tc_copy(x2d):
  rows, cols = x2d.shape
  blk = 512
  return pl.pallas_call(
      lambda x_ref, o_ref: o_ref.__setitem__((...,), x_ref[...]),
      out_shape=jax.ShapeDtypeStruct((rows, cols), x2d.dtype),
      grid=(rows // blk,),
      in_specs=[pl.BlockSpec((blk, cols), lambda i: (i, 0))],
      out_specs=pl.BlockSpec((blk, cols), lambda i: (i, 0)),
  )(x2d)


def _make_sc_scatter(total, k_rows):
  mesh = plsc.VectorSubcoreMesh(core_axis_name="c", subcore_axis_name="s")

  @functools.partial(
      pl.kernel,
      mesh=mesh,
      scratch_types=[
          pltpu.VMEM((k_rows, BATCH), jnp.int32),
          pltpu.VMEM((k_rows, BATCH), jnp.int32),
          pltpu.VMEM((k_rows, BATCH), jnp.float32),
          pltpu.VMEM((2, LANES), jnp.float32),
          pltpu.SemaphoreType.DMA,
      ],
  )
  def sc_scatter(out_ref, idx_hbm, rand_hbm, sz_hbm, idx_v, gat_v, val_v,
                 sz_v, sem):
    wid = lax.axis_index("s") * NC + lax.axis_index("c")
    pltpu.sync_copy(sz_hbm, sz_v)
    pltpu.sync_copy(idx_hbm.at[wid], idx_v)
    scale = sz_v[0, :]
    zp = sz_v[1, :]

    @pl.loop(0, k_rows)
    def _row(r):
      pltpu.async_copy(rand_hbm.at[idx_v.at[r]], gat_v.at[r], sem).wait()
      for k in range(BATCH // LANES):
        x = gat_v[r, pl.ds(k * LANES, LANES)]
        val_v[r, pl.ds(k * LANES, LANES)] = (
            x.astype(jnp.float32) - zp) * scale
      pltpu.async_copy(val_v.at[r], out_ref.at[idx_v.at[r]], sem).wait()

  return sc_scatter


def kernel(value, scale, zero_point, flat_indices, rand_int):
  shape = value.shape
  total = value.size
  n_idx = flat_indices.shape[0]

  k_rows = -(-n_idx // (NW * BATCH))
  n_pad = NW * k_rows * BATCH
  pad = jnp.broadcast_to(flat_indices[:1], (n_pad - n_idx,))
  idx3 = jnp.concatenate([flat_indices, pad]).reshape(NW, k_rows, BATCH)

  sz = jnp.stack([
      jnp.broadcast_to(scale.astype(jnp.float32), (LANES,)),
      jnp.broadcast_to(zero_point.astype(jnp.float32), (LANES,)),
  ])

  out2d = _tc_copy(value.reshape(-1, 2048))
  out_ref = jax.new_ref(out2d.reshape(total))
  _make_sc_scatter(total, k_rows)(
      out_ref, idx3, rand_int.reshape(total), sz)
  return out_ref[...].reshape(shape)
